# Initial kernel scaffold; baseline (speedup 1.0000x reference)
import jax
import jax.numpy as jnp
from jax.experimental import pallas as pl


def kernel(hidden_states, W):
    raise NotImplementedError("write your pallas kernel here")


# speedup vs baseline: 1.1177x; 1.1177x over previous
import functools

import jax
import jax.numpy as jnp
from jax.experimental import pallas as pl

_TOP_K = 8
_BT = 512


def _fused_body(x_ref, w_ref, logits_ref, rw_ref, idx_ref):
    x = x_ref[...]
    w = w_ref[...]
    logits = jax.lax.dot_general(
        x, w, (((1,), (1,)), ((), ())), preferred_element_type=jnp.float32
    )
    logits_ref[...] = logits

    col = jax.lax.broadcasted_iota(jnp.int32, logits.shape, 1)
    n_experts = logits.shape[1]
    work = logits
    vals, inds = [], []
    for _ in range(_TOP_K):
        m = jnp.max(work, axis=1, keepdims=True)
        am = jnp.min(
            jnp.where(work == m, col, n_experts), axis=1, keepdims=True
        )
        vals.append(m)
        inds.append(am)
        work = jnp.where(col == am, -jnp.inf, work)

    v = jnp.concatenate(vals, axis=1)
    e = jnp.exp(v - vals[0])
    rw_ref[...] = e / jnp.sum(e, axis=1, keepdims=True)
    idx_ref[...] = jnp.concatenate(inds, axis=1)


@functools.partial(jax.jit, static_argnames=())
def kernel(hidden_states, W):
    n_tokens, hidden_dim = hidden_states.shape
    n_experts = W.shape[0]
    grid = (n_tokens // _BT,)
    logits, rw, idx = pl.pallas_call(
        _fused_body,
        grid=grid,
        in_specs=[
            pl.BlockSpec((_BT, hidden_dim), lambda i: (i, 0)),
            pl.BlockSpec((n_experts, hidden_dim), lambda i: (0, 0)),
        ],
        out_specs=[
            pl.BlockSpec((_BT, n_experts), lambda i: (i, 0)),
            pl.BlockSpec((_BT, _TOP_K), lambda i: (i, 0)),
            pl.BlockSpec((_BT, _TOP_K), lambda i: (i, 0)),
        ],
        out_shape=[
            jax.ShapeDtypeStruct((n_tokens, n_experts), jnp.float32),
            jax.ShapeDtypeStruct((n_tokens, _TOP_K), jnp.float32),
            jax.ShapeDtypeStruct((n_tokens, _TOP_K), jnp.int32),
        ],
    )(hidden_states, W)
    return rw, idx, logits
